# baseline (speedup 1.0000x reference)
import functools
import math

import jax
import jax.numpy as jnp
from jax import lax
from jax.experimental import pallas as pl
from jax.experimental.pallas import tpu as pltpu
from jax.experimental.pallas import tpu_sc as plsc

N = 10000
E = 320000
D = 128

NC = 2
NS = 16
NW = NC * NS
CH = 64

EW = -(-(-(-E // NW)) // 128) * 128
CHUNKS = EW // CH
EP = NW * EW
NPAD = 10112
RPT = NPAD // NS

M0 = 104
M1 = 2 * CHUNKS - M0

CHC = 128
CCHUNKS = EW // CHC


def _fill_const(ref, rows, width, val):
    def _f(i, _):
        r = i // (width // 16)
        k = i % (width // 16)
        ref[r, pl.ds(k * 16, 16)] = jnp.full((16,), val, jnp.float32)
        return ()
    lax.fori_loop(0, rows * (width // 16), _f, ())


def _sc_agg_body(x_hbm, src_hbm, dst_hbm, agg_out,
                 sv0, sv1, dv0, dv1, rows0, rows1, agg_s,
                 semA, semB, semS0, semS1, semD0, semD1):
    c = lax.axis_index("c")
    s = lax.axis_index("s")

    cbase = jnp.where(c == 0, s * M0, NS * M0 + s * M1)
    m = jnp.where(c == 0, M0, M1)

    _fill_const(rows0, CH, D, 0.0)

    r0 = s * RPT
    for off in range(0, RPT, CH):
        sz = min(CH, RPT - off)
        pltpu.sync_copy(rows0.at[pl.ds(0, sz)], agg_s.at[pl.ds(r0 + off, sz)])

    plsc.subcore_barrier()

    def _iload(j, iv, sem, hbm):
        pltpu.async_copy(hbm.at[pl.ds((cbase + j) * CH, CH)], iv, sem)

    def _iwait(iv, sem, hbm):
        pltpu.make_async_copy(hbm.at[pl.ds(0, CH)], iv, sem).wait()

    def _start(sv, rows, sem):
        pltpu.async_copy(x_hbm.at[sv], rows, sem)

    def _wait(rows, sem):
        pltpu.make_async_copy(x_hbm.at[pl.ds(0, CH)], rows, sem).wait()

    def _scat(rows, dv):
        pltpu.sync_copy(rows, agg_s.at[dv], add=True)

    pltpu.sync_copy(src_hbm.at[pl.ds(cbase * CH, CH)], sv0)
    _iload(0, dv0, semD0, dst_hbm)
    _iload(1, sv1, semS1, src_hbm)
    _iload(1, dv1, semD1, dst_hbm)
    _start(sv0, rows0, semA)

    def _pair(i, _):
        k0 = 2 * i
        _iwait(sv1, semS1, src_hbm)
        _start(sv1, rows1, semB)
        _wait(rows0, semA)
        _iload(k0 + 2, sv0, semS0, src_hbm)
        _iwait(dv0, semD0, dst_hbm)
        _scat(rows0, dv0)
        _iload(k0 + 2, dv0, semD0, dst_hbm)
        _iwait(sv0, semS0, src_hbm)
        _start(sv0, rows0, semA)
        _wait(rows1, semB)
        _iload(k0 + 3, sv1, semS1, src_hbm)
        _iwait(dv1, semD1, dst_hbm)
        _scat(rows1, dv1)
        _iload(k0 + 3, dv1, semD1, dst_hbm)
        return ()
    lax.fori_loop(0, m // 2 - 1, _pair, ())

    _iwait(sv1, semS1, src_hbm)
    _start(sv1, rows1, semB)
    _wait(rows0, semA)
    _iwait(dv0, semD0, dst_hbm)
    _scat(rows0, dv0)
    _wait(rows1, semB)
    _iwait(dv1, semD1, dst_hbm)
    _scat(rows1, dv1)

    plsc.subcore_barrier()

    pltpu.sync_copy(agg_s.at[pl.ds(r0, RPT)], agg_out.at[c, pl.ds(r0, RPT)])


def _sc_cnt_body(dst_hbm, cnt_out, dst_big, ones_v, cnt_s, sem):
    c = lax.axis_index("c")
    s = lax.axis_index("s")
    wid = c * NS + s

    _fill_const(ones_v, CHC, D, 0.0)
    r0 = s * RPT
    for off in range(0, RPT, CHC):
        sz = min(CHC, RPT - off)
        pltpu.sync_copy(ones_v.at[pl.ds(0, sz)], cnt_s.at[pl.ds(r0 + off, sz)])
    _fill_const(ones_v, CHC, D, 1.0)

    pltpu.sync_copy(dst_hbm.at[wid], dst_big)

    plsc.subcore_barrier()

    def _fire(k, _):
        pltpu.async_copy(ones_v, cnt_s.at[dst_big.at[k]], sem, add=True)
        return ()
    lax.fori_loop(0, CCHUNKS, _fire, ())

    def _drain(k, _):
        pltpu.make_async_copy(ones_v, cnt_s.at[pl.ds(0, CHC)], sem).wait()
        return ()
    lax.fori_loop(0, CCHUNKS, _drain, ())

    plsc.subcore_barrier()

    pltpu.sync_copy(cnt_s.at[pl.ds(r0, RPT)], cnt_out.at[c, pl.ds(r0, RPT)])


@functools.cache
def _get_sc_agg():
    return pl.kernel(
        _sc_agg_body,
        out_type=jax.ShapeDtypeStruct((NC, NPAD, D), jnp.float32),
        mesh=plsc.VectorSubcoreMesh(core_axis_name="c", subcore_axis_name="s",
                                    num_cores=NC, num_subcores=NS),
        scratch_types=[
            pltpu.VMEM((CH,), jnp.int32),
            pltpu.VMEM((CH,), jnp.int32),
            pltpu.VMEM((CH,), jnp.int32),
            pltpu.VMEM((CH,), jnp.int32),
            pltpu.VMEM((CH, D), jnp.float32),
            pltpu.VMEM((CH, D), jnp.float32),
            pltpu.VMEM_SHARED((NPAD, D), jnp.float32),
            pltpu.SemaphoreType.DMA,
            pltpu.SemaphoreType.DMA,
            pltpu.SemaphoreType.DMA,
            pltpu.SemaphoreType.DMA,
            pltpu.SemaphoreType.DMA,
            pltpu.SemaphoreType.DMA,
        ],
    )


@functools.cache
def _get_sc_cnt():
    return pl.kernel(
        _sc_cnt_body,
        out_type=jax.ShapeDtypeStruct((NC, NPAD, D), jnp.float32),
        mesh=plsc.VectorSubcoreMesh(core_axis_name="c", subcore_axis_name="s",
                                    num_cores=NC, num_subcores=NS),
        scratch_types=[
            pltpu.VMEM((CCHUNKS, CHC), jnp.int32),
            pltpu.VMEM((CHC, D), jnp.float32),
            pltpu.VMEM_SHARED((NPAD, D), jnp.float32),
            pltpu.SemaphoreType.DMA,
        ],
    )


_SQRT2 = math.sqrt(2.0)
_BLK = 400
_GRID = N // _BLK


def _tc_dense_body(x_ref, agg_ref, cnt_ref, wl_ref, bl_ref, wr_ref,
                   g_ref, b_ref, o_ref):
    xb = x_ref[...]
    agg = agg_ref[0] + agg_ref[1]
    cnt = cnt_ref[0, :, 0:1] + cnt_ref[1, :, 0:1]
    mean = agg / jnp.maximum(cnt, 1.0)
    h = (jnp.dot(mean, wl_ref[...], preferred_element_type=jnp.float32)
         + bl_ref[...]
         + jnp.dot(xb, wr_ref[...], preferred_element_type=jnp.float32))
    mu = jnp.mean(h, axis=1, keepdims=True)
    var = jnp.mean((h - mu) ** 2, axis=1, keepdims=True)
    hn = (h - mu) * lax.rsqrt(var + 1e-5) * g_ref[...] + b_ref[...]
    gelu = 0.5 * hn * (1.0 + lax.erf(hn / _SQRT2))
    o_ref[...] = xb + gelu


def _tc_dense(x, agg, cnt, Wl, bl, Wr, g, b):
    return pl.pallas_call(
        _tc_dense_body,
        grid=(_GRID,),
        in_specs=[
            pl.BlockSpec((_BLK, D), lambda i: (i, 0)),
            pl.BlockSpec((NC, _BLK, D), lambda i: (0, i, 0)),
            pl.BlockSpec((NC, _BLK, D), lambda i: (0, i, 0)),
            pl.BlockSpec((D, D), lambda i: (0, 0)),
            pl.BlockSpec((1, D), lambda i: (0, 0)),
            pl.BlockSpec((D, D), lambda i: (0, 0)),
            pl.BlockSpec((1, D), lambda i: (0, 0)),
            pl.BlockSpec((1, D), lambda i: (0, 0)),
        ],
        out_specs=pl.BlockSpec((_BLK, D), lambda i: (i, 0)),
        out_shape=jax.ShapeDtypeStruct((N, D), jnp.float32),
    )(x, agg, cnt, Wl, bl.reshape(1, D), Wr, g.reshape(1, D), b.reshape(1, D))


def kernel(x, edge_index, Wl1, bl1, Wr1, g1, b1, Wl2, bl2, Wr2, g2, b2):
    src = edge_index[0].astype(jnp.int32)
    dst = edge_index[1].astype(jnp.int32)
    pad = EP - E
    srcp = jnp.concatenate([src, jnp.zeros((pad,), jnp.int32)])
    dstp = jnp.concatenate([dst, jnp.full((pad,), N, jnp.int32)])
    dst3c = dstp.reshape(NW, CCHUNKS, CHC)

    cnt = _get_sc_cnt()(dst3c)
    sc_agg = _get_sc_agg()
    agg1 = sc_agg(x, srcp, dstp)
    x1 = _tc_dense(x, agg1, cnt, Wl1, bl1, Wr1, g1, b1)
    agg2 = sc_agg(x1, srcp, dstp)
    x2 = _tc_dense(x1, agg2, cnt, Wl2, bl2, Wr2, g2, b2)
    return x2


# speedup vs baseline: 1.2074x; 1.2074x over previous
import functools
import math

import jax
import jax.numpy as jnp
from jax import lax
from jax.experimental import pallas as pl
from jax.experimental.pallas import tpu as pltpu
from jax.experimental.pallas import tpu_sc as plsc

N = 10000
E = 320000
D = 128

NC = 2
NS = 16
NW = NC * NS
CH = 64

EW = -(-(-(-E // NW)) // 128) * 128
CHUNKS = EW // CH
EP = NW * EW
NPAD = 10112
RPT = NPAD // NS

M0 = 212
M1 = 2 * CHUNKS - M0

CHC = 128
CCHUNKS = EW // CHC


def _fill_const(ref, rows, width, val):
    def _f(i, _):
        r = i // (width // 16)
        k = i % (width // 16)
        ref[r, pl.ds(k * 16, 16)] = jnp.full((16,), val, jnp.float32)
        return ()
    lax.fori_loop(0, rows * (width // 16), _f, ())


def _sc_agg_body(x_hbm, src_hbm, dst_hbm, agg_out,
                 sv0, sv1, dv0, dv1, rows0, rows1, agg_s,
                 semA, semB, semS0, semS1, semD0, semD1):
    c = lax.axis_index("c")
    s = lax.axis_index("s")

    cbase = jnp.where(c == 0, s * M0, NS * M0 + s * M1)
    m = jnp.where(c == 0, M0, M1)

    _fill_const(rows0, CH, D, 0.0)

    r0 = s * RPT
    for off in range(0, RPT, CH):
        sz = min(CH, RPT - off)
        pltpu.sync_copy(rows0.at[pl.ds(0, sz)], agg_s.at[pl.ds(r0 + off, sz)])

    plsc.subcore_barrier()

    def _iload(j, iv, sem, hbm):
        pltpu.async_copy(hbm.at[pl.ds((cbase + j) * CH, CH)], iv, sem)

    def _iwait(iv, sem, hbm):
        pltpu.make_async_copy(hbm.at[pl.ds(0, CH)], iv, sem).wait()

    def _start(sv, rows, sem):
        pltpu.async_copy(x_hbm.at[sv], rows, sem)

    def _wait(rows, sem):
        pltpu.make_async_copy(x_hbm.at[pl.ds(0, CH)], rows, sem).wait()

    def _scat(rows, dv):
        pltpu.sync_copy(rows, agg_s.at[dv], add=True)

    pltpu.sync_copy(src_hbm.at[pl.ds(cbase * CH, CH)], sv0)
    _iload(0, dv0, semD0, dst_hbm)
    _iload(1, sv1, semS1, src_hbm)
    _iload(1, dv1, semD1, dst_hbm)
    _start(sv0, rows0, semA)

    def _pair(i, _):
        k0 = 2 * i
        _iwait(sv1, semS1, src_hbm)
        _start(sv1, rows1, semB)
        _wait(rows0, semA)
        _iload(k0 + 2, sv0, semS0, src_hbm)
        _iwait(dv0, semD0, dst_hbm)
        _scat(rows0, dv0)
        _iload(k0 + 2, dv0, semD0, dst_hbm)
        _iwait(sv0, semS0, src_hbm)
        _start(sv0, rows0, semA)
        _wait(rows1, semB)
        _iload(k0 + 3, sv1, semS1, src_hbm)
        _iwait(dv1, semD1, dst_hbm)
        _scat(rows1, dv1)
        _iload(k0 + 3, dv1, semD1, dst_hbm)
        return ()
    lax.fori_loop(0, m // 2 - 1, _pair, ())

    _iwait(sv1, semS1, src_hbm)
    _start(sv1, rows1, semB)
    _wait(rows0, semA)
    _iwait(dv0, semD0, dst_hbm)
    _scat(rows0, dv0)
    _wait(rows1, semB)
    _iwait(dv1, semD1, dst_hbm)
    _scat(rows1, dv1)

    plsc.subcore_barrier()

    pltpu.sync_copy(agg_s.at[pl.ds(r0, RPT)], agg_out.at[c, pl.ds(r0, RPT)])


def _sc_cnt_body(dst_hbm, cnt_out, dst_big, ones_v, cnt_s, sem):
    c = lax.axis_index("c")
    s = lax.axis_index("s")
    wid = c * NS + s

    _fill_const(ones_v, CHC, D, 0.0)
    r0 = s * RPT
    for off in range(0, RPT, CHC):
        sz = min(CHC, RPT - off)
        pltpu.sync_copy(ones_v.at[pl.ds(0, sz)], cnt_s.at[pl.ds(r0 + off, sz)])
    _fill_const(ones_v, CHC, D, 1.0)

    pltpu.sync_copy(dst_hbm.at[wid], dst_big)

    plsc.subcore_barrier()

    def _fire(k, _):
        pltpu.async_copy(ones_v, cnt_s.at[dst_big.at[k]], sem, add=True)
        return ()
    lax.fori_loop(0, CCHUNKS, _fire, ())

    def _drain(k, _):
        pltpu.make_async_copy(ones_v, cnt_s.at[pl.ds(0, CHC)], sem).wait()
        return ()
    lax.fori_loop(0, CCHUNKS, _drain, ())

    plsc.subcore_barrier()

    pltpu.sync_copy(cnt_s.at[pl.ds(r0, RPT)], cnt_out.at[c, pl.ds(r0, RPT)])


@functools.cache
def _get_sc_agg():
    return pl.kernel(
        _sc_agg_body,
        out_type=jax.ShapeDtypeStruct((NC, NPAD, D), jnp.float32),
        mesh=plsc.VectorSubcoreMesh(core_axis_name="c", subcore_axis_name="s",
                                    num_cores=NC, num_subcores=NS),
        scratch_types=[
            pltpu.VMEM((CH,), jnp.int32),
            pltpu.VMEM((CH,), jnp.int32),
            pltpu.VMEM((CH,), jnp.int32),
            pltpu.VMEM((CH,), jnp.int32),
            pltpu.VMEM((CH, D), jnp.float32),
            pltpu.VMEM((CH, D), jnp.float32),
            pltpu.VMEM_SHARED((NPAD, D), jnp.float32),
            pltpu.SemaphoreType.DMA,
            pltpu.SemaphoreType.DMA,
            pltpu.SemaphoreType.DMA,
            pltpu.SemaphoreType.DMA,
            pltpu.SemaphoreType.DMA,
            pltpu.SemaphoreType.DMA,
        ],
    )


@functools.cache
def _get_sc_cnt():
    return pl.kernel(
        _sc_cnt_body,
        out_type=jax.ShapeDtypeStruct((NC, NPAD, D), jnp.float32),
        mesh=plsc.VectorSubcoreMesh(core_axis_name="c", subcore_axis_name="s",
                                    num_cores=NC, num_subcores=NS),
        scratch_types=[
            pltpu.VMEM((CCHUNKS, CHC), jnp.int32),
            pltpu.VMEM((CHC, D), jnp.float32),
            pltpu.VMEM_SHARED((NPAD, D), jnp.float32),
            pltpu.SemaphoreType.DMA,
        ],
    )


_SQRT2 = math.sqrt(2.0)
_BLK = 400
_GRID = N // _BLK


def _tc_dense_body(x_ref, agg_ref, cnt_ref, wl_ref, bl_ref, wr_ref,
                   g_ref, b_ref, o_ref):
    xb = x_ref[...]
    agg = agg_ref[0] + agg_ref[1]
    cnt = cnt_ref[0, :, 0:1] + cnt_ref[1, :, 0:1]
    mean = agg / jnp.maximum(cnt, 1.0)
    h = (jnp.dot(mean, wl_ref[...], preferred_element_type=jnp.float32)
         + bl_ref[...]
         + jnp.dot(xb, wr_ref[...], preferred_element_type=jnp.float32))
    mu = jnp.mean(h, axis=1, keepdims=True)
    var = jnp.mean((h - mu) ** 2, axis=1, keepdims=True)
    hn = (h - mu) * lax.rsqrt(var + 1e-5) * g_ref[...] + b_ref[...]
    gelu = 0.5 * hn * (1.0 + lax.erf(hn / _SQRT2))
    o_ref[...] = xb + gelu


def _tc_dense(x, agg, cnt, Wl, bl, Wr, g, b):
    return pl.pallas_call(
        _tc_dense_body,
        grid=(_GRID,),
        in_specs=[
            pl.BlockSpec((_BLK, D), lambda i: (i, 0)),
            pl.BlockSpec((NC, _BLK, D), lambda i: (0, i, 0)),
            pl.BlockSpec((NC, _BLK, D), lambda i: (0, i, 0)),
            pl.BlockSpec((D, D), lambda i: (0, 0)),
            pl.BlockSpec((1, D), lambda i: (0, 0)),
            pl.BlockSpec((D, D), lambda i: (0, 0)),
            pl.BlockSpec((1, D), lambda i: (0, 0)),
            pl.BlockSpec((1, D), lambda i: (0, 0)),
        ],
        out_specs=pl.BlockSpec((_BLK, D), lambda i: (i, 0)),
        out_shape=jax.ShapeDtypeStruct((N, D), jnp.float32),
    )(x, agg, cnt, Wl, bl.reshape(1, D), Wr, g.reshape(1, D), b.reshape(1, D))


def kernel(x, edge_index, Wl1, bl1, Wr1, g1, b1, Wl2, bl2, Wr2, g2, b2):
    src = edge_index[0].astype(jnp.int32)
    dst = edge_index[1].astype(jnp.int32)
    pad = EP - E
    srcp = jnp.concatenate([src, jnp.zeros((pad,), jnp.int32)])
    dstp = jnp.concatenate([dst, jnp.full((pad,), N, jnp.int32)])
    dst3c = dstp.reshape(NW, CCHUNKS, CHC)

    cnt = _get_sc_cnt()(dst3c)
    sc_agg = _get_sc_agg()
    agg1 = sc_agg(x, srcp, dstp)
    x1 = _tc_dense(x, agg1, cnt, Wl1, bl1, Wr1, g1, b1)
    agg2 = sc_agg(x1, srcp, dstp)
    x2 = _tc_dense(x1, agg2, cnt, Wl2, bl2, Wr2, g2, b2)
    return x2


# speedup vs baseline: 1.4501x; 1.2009x over previous
import functools
import math

import jax
import jax.numpy as jnp
from jax import lax
from jax.experimental import pallas as pl
from jax.experimental.pallas import tpu as pltpu
from jax.experimental.pallas import tpu_sc as plsc

N = 10000
E = 320000
D = 128

NC = 2
NS = 16
NW = NC * NS
CH = 96

EW = -(-(-(-E // NW)) // CH) * CH
CHUNKS = EW // CH
EP = NW * EW
NPAD = 10112
RPT = NPAD // NS

M0 = 2 * ((2 * CHUNKS * 2) // 3 // 2)
M1 = 2 * CHUNKS - M0

CHC = 128
EWC = -(-(-(-E // NW)) // CHC) * CHC
EPC = NW * EWC
CCHUNKS = EWC // CHC


def _fill_const(ref, rows, width, val):
    def _f(i, _):
        r = i // (width // 16)
        k = i % (width // 16)
        ref[r, pl.ds(k * 16, 16)] = jnp.full((16,), val, jnp.float32)
        return ()
    lax.fori_loop(0, rows * (width // 16), _f, ())


def _sc_agg_body(x_hbm, src_hbm, dst_hbm, agg_out,
                 sv0, sv1, dv0, dv1, rows0, rows1, agg_s,
                 semA, semB, semS0, semS1, semD0, semD1):
    c = lax.axis_index("c")
    s = lax.axis_index("s")

    cbase = jnp.where(c == 0, s * M0, NS * M0 + s * M1)
    m = jnp.where(c == 0, M0, M1)

    _fill_const(rows0, CH, D, 0.0)

    r0 = s * RPT
    for off in range(0, RPT, CH):
        sz = min(CH, RPT - off)
        pltpu.sync_copy(rows0.at[pl.ds(0, sz)], agg_s.at[pl.ds(r0 + off, sz)])

    plsc.subcore_barrier()

    def _iload(j, iv, sem, hbm):
        pltpu.async_copy(hbm.at[pl.ds((cbase + j) * CH, CH)], iv, sem)

    def _iwait(iv, sem, hbm):
        pltpu.make_async_copy(hbm.at[pl.ds(0, CH)], iv, sem).wait()

    def _start(sv, rows, sem):
        pltpu.async_copy(x_hbm.at[sv], rows, sem)

    def _wait(rows, sem):
        pltpu.make_async_copy(x_hbm.at[pl.ds(0, CH)], rows, sem).wait()

    def _scat(rows, dv):
        pltpu.sync_copy(rows, agg_s.at[dv], add=True)

    pltpu.sync_copy(src_hbm.at[pl.ds(cbase * CH, CH)], sv0)
    _iload(0, dv0, semD0, dst_hbm)
    _iload(1, sv1, semS1, src_hbm)
    _iload(1, dv1, semD1, dst_hbm)
    _start(sv0, rows0, semA)

    def _pair(i, _):
        k0 = 2 * i
        _iwait(sv1, semS1, src_hbm)
        _start(sv1, rows1, semB)
        _wait(rows0, semA)
        _iload(k0 + 2, sv0, semS0, src_hbm)
        _iwait(dv0, semD0, dst_hbm)
        _scat(rows0, dv0)
        _iload(k0 + 2, dv0, semD0, dst_hbm)
        _iwait(sv0, semS0, src_hbm)
        _start(sv0, rows0, semA)
        _wait(rows1, semB)
        _iload(k0 + 3, sv1, semS1, src_hbm)
        _iwait(dv1, semD1, dst_hbm)
        _scat(rows1, dv1)
        _iload(k0 + 3, dv1, semD1, dst_hbm)
        return ()
    lax.fori_loop(0, m // 2 - 1, _pair, ())

    _iwait(sv1, semS1, src_hbm)
    _start(sv1, rows1, semB)
    _wait(rows0, semA)
    _iwait(dv0, semD0, dst_hbm)
    _scat(rows0, dv0)
    _wait(rows1, semB)
    _iwait(dv1, semD1, dst_hbm)
    _scat(rows1, dv1)

    plsc.subcore_barrier()

    pltpu.sync_copy(agg_s.at[pl.ds(r0, RPT)], agg_out.at[c, pl.ds(r0, RPT)])


def _sc_cnt_body(dst_hbm, cnt_out, dst_big, ones_v, cnt_s, sem):
    c = lax.axis_index("c")
    s = lax.axis_index("s")
    wid = c * NS + s

    _fill_const(ones_v, CHC, D, 0.0)
    r0 = s * RPT
    for off in range(0, RPT, CHC):
        sz = min(CHC, RPT - off)
        pltpu.sync_copy(ones_v.at[pl.ds(0, sz)], cnt_s.at[pl.ds(r0 + off, sz)])
    _fill_const(ones_v, CHC, D, 1.0)

    pltpu.sync_copy(dst_hbm.at[wid], dst_big)

    plsc.subcore_barrier()

    def _fire(k, _):
        pltpu.async_copy(ones_v, cnt_s.at[dst_big.at[k]], sem, add=True)
        return ()
    lax.fori_loop(0, CCHUNKS, _fire, ())

    def _drain(k, _):
        pltpu.make_async_copy(ones_v, cnt_s.at[pl.ds(0, CHC)], sem).wait()
        return ()
    lax.fori_loop(0, CCHUNKS, _drain, ())

    plsc.subcore_barrier()

    pltpu.sync_copy(cnt_s.at[pl.ds(r0, RPT)], cnt_out.at[c, pl.ds(r0, RPT)])


@functools.cache
def _get_sc_agg():
    return pl.kernel(
        _sc_agg_body,
        out_type=jax.ShapeDtypeStruct((NC, NPAD, D), jnp.float32),
        mesh=plsc.VectorSubcoreMesh(core_axis_name="c", subcore_axis_name="s",
                                    num_cores=NC, num_subcores=NS),
        scratch_types=[
            pltpu.VMEM((CH,), jnp.int32),
            pltpu.VMEM((CH,), jnp.int32),
            pltpu.VMEM((CH,), jnp.int32),
            pltpu.VMEM((CH,), jnp.int32),
            pltpu.VMEM((CH, D), jnp.float32),
            pltpu.VMEM((CH, D), jnp.float32),
            pltpu.VMEM_SHARED((NPAD, D), jnp.float32),
            pltpu.SemaphoreType.DMA,
            pltpu.SemaphoreType.DMA,
            pltpu.SemaphoreType.DMA,
            pltpu.SemaphoreType.DMA,
            pltpu.SemaphoreType.DMA,
            pltpu.SemaphoreType.DMA,
        ],
    )


@functools.cache
def _get_sc_cnt():
    return pl.kernel(
        _sc_cnt_body,
        out_type=jax.ShapeDtypeStruct((NC, NPAD, D), jnp.float32),
        mesh=plsc.VectorSubcoreMesh(core_axis_name="c", subcore_axis_name="s",
                                    num_cores=NC, num_subcores=NS),
        scratch_types=[
            pltpu.VMEM((CCHUNKS, CHC), jnp.int32),
            pltpu.VMEM((CHC, D), jnp.float32),
            pltpu.VMEM_SHARED((NPAD, D), jnp.float32),
            pltpu.SemaphoreType.DMA,
        ],
    )


_SQRT2 = math.sqrt(2.0)
_BLK = 400
_GRID = N // _BLK


def _tc_dense_body(x_ref, agg_ref, cnt_ref, wl_ref, bl_ref, wr_ref,
                   g_ref, b_ref, o_ref):
    xb = x_ref[...]
    agg = agg_ref[0] + agg_ref[1]
    cnt = cnt_ref[0, :, 0:1] + cnt_ref[1, :, 0:1]
    mean = agg / jnp.maximum(cnt, 1.0)
    h = (jnp.dot(mean, wl_ref[...], preferred_element_type=jnp.float32)
         + bl_ref[...]
         + jnp.dot(xb, wr_ref[...], preferred_element_type=jnp.float32))
    mu = jnp.mean(h, axis=1, keepdims=True)
    var = jnp.mean((h - mu) ** 2, axis=1, keepdims=True)
    hn = (h - mu) * lax.rsqrt(var + 1e-5) * g_ref[...] + b_ref[...]
    gelu = 0.5 * hn * (1.0 + lax.erf(hn / _SQRT2))
    o_ref[...] = xb + gelu


def _tc_dense(x, agg, cnt, Wl, bl, Wr, g, b):
    return pl.pallas_call(
        _tc_dense_body,
        grid=(_GRID,),
        in_specs=[
            pl.BlockSpec((_BLK, D), lambda i: (i, 0)),
            pl.BlockSpec((NC, _BLK, D), lambda i: (0, i, 0)),
            pl.BlockSpec((NC, _BLK, D), lambda i: (0, i, 0)),
            pl.BlockSpec((D, D), lambda i: (0, 0)),
            pl.BlockSpec((1, D), lambda i: (0, 0)),
            pl.BlockSpec((D, D), lambda i: (0, 0)),
            pl.BlockSpec((1, D), lambda i: (0, 0)),
            pl.BlockSpec((1, D), lambda i: (0, 0)),
        ],
        out_specs=pl.BlockSpec((_BLK, D), lambda i: (i, 0)),
        out_shape=jax.ShapeDtypeStruct((N, D), jnp.float32),
    )(x, agg, cnt, Wl, bl.reshape(1, D), Wr, g.reshape(1, D), b.reshape(1, D))


def kernel(x, edge_index, Wl1, bl1, Wr1, g1, b1, Wl2, bl2, Wr2, g2, b2):
    src = edge_index[0].astype(jnp.int32)
    dst = edge_index[1].astype(jnp.int32)
    srcp = jnp.concatenate([src, jnp.zeros((EP - E,), jnp.int32)])
    dstp = jnp.concatenate([dst, jnp.full((EP - E,), N, jnp.int32)])
    dstc = jnp.concatenate([dst, jnp.full((EPC - E,), N, jnp.int32)])
    dst3c = dstc.reshape(NW, CCHUNKS, CHC)

    cnt = _get_sc_cnt()(dst3c)
    sc_agg = _get_sc_agg()
    agg1 = sc_agg(x, srcp, dstp)
    x1 = _tc_dense(x, agg1, cnt, Wl1, bl1, Wr1, g1, b1)
    agg2 = sc_agg(x1, srcp, dstp)
    x2 = _tc_dense(x1, agg2, cnt, Wl2, bl2, Wr2, g2, b2)
    return x2
